# Initial kernel scaffold; baseline (speedup 1.0000x reference)
import jax
import jax.numpy as jnp
from jax.experimental import pallas as pl


def kernel(x, weight):
    raise NotImplementedError("write your pallas kernel here")


# speedup vs baseline: 1.8783x; 1.8783x over previous
import functools

import jax
import jax.numpy as jnp
from jax import lax
from jax.experimental import pallas as pl
from jax.experimental.pallas import tpu as pltpu
from jax.experimental.pallas import tpu_sc as plsc

_NUM_WORKERS = 32
_CHUNK = 128
_NBUF = 4


def _embedding_body(nchunks, chunk, x_hbm, w_hbm, out_hbm, idx_v, rows_v,
                    *gsems):
    nbuf = len(gsems)
    b_per_w = nchunks * chunk
    wid = lax.axis_index("s") * 2 + lax.axis_index("c")
    base = pl.multiple_of(wid * b_per_w, b_per_w)

    pltpu.sync_copy(x_hbm.at[pl.ds(base, b_per_w)], idx_v)

    for b in range(nbuf):
        pltpu.async_copy(
            w_hbm.at[idx_v.at[pl.ds(b * chunk, chunk)]], rows_v.at[b],
            gsems[b])

    def group(g, carry):
        for b in range(nbuf):
            i = g * nbuf + b
            off = pl.multiple_of(i * chunk, chunk)
            pltpu.make_async_copy(
                w_hbm.at[idx_v.at[pl.ds(off, chunk)]], rows_v.at[b],
                gsems[b]).wait()
            pltpu.sync_copy(rows_v.at[b], out_hbm.at[pl.ds(base + off, chunk)])

            nxt = i + nbuf

            @pl.when(nxt < nchunks)
            def _():
                noff = pl.multiple_of(nxt * chunk, chunk)
                pltpu.async_copy(
                    w_hbm.at[idx_v.at[pl.ds(noff, chunk)]], rows_v.at[b],
                    gsems[b])

        return carry

    lax.fori_loop(0, nchunks // nbuf, group, 0)


@jax.jit
def kernel(x, weight):
    batch, hist = x.shape
    n = batch * hist
    dim = weight.shape[1]
    assert n % (_NUM_WORKERS * _CHUNK) == 0
    b_per_w = n // _NUM_WORKERS
    nchunks = b_per_w // _CHUNK

    idx = x.astype(jnp.int32).reshape(n)

    mesh = plsc.VectorSubcoreMesh(core_axis_name="c", subcore_axis_name="s")
    scratch = [
        pltpu.VMEM((b_per_w,), jnp.int32),
        pltpu.VMEM((_NBUF, _CHUNK, dim), jnp.float32),
    ] + [pltpu.SemaphoreType.DMA] * _NBUF

    out = pl.kernel(
        functools.partial(_embedding_body, nchunks, _CHUNK),
        out_type=jax.ShapeDtypeStruct((n, dim), jnp.float32),
        mesh=mesh,
        scratch_types=scratch,
        compiler_params=pltpu.CompilerParams(use_tc_tiling_on_sc=False),
    )(idx, weight)
    return out.reshape(batch, hist, dim)
